# Initial kernel scaffold; baseline (speedup 1.0000x reference)
import jax
import jax.numpy as jnp
from jax.experimental import pallas as pl


def kernel(x, W_mlp, b_mlp, Wq0, Wk0, Wv0, Ws0, Wq1, Wk1, Wv1, Ws1, W_out, b_out, pool_w):
    raise NotImplementedError("write your pallas kernel here")


# speedup vs baseline: 53.0728x; 53.0728x over previous
import functools

import numpy as np
import jax
import jax.numpy as jnp
from jax import lax
from jax.experimental import pallas as pl
from jax.experimental.pallas import tpu as pltpu
from jax.experimental.pallas import tpu_sc as plsc

_PIECES = 16
_T = 64
_KSEL = 7
_HEADS = 8
_HD = 32
_HID = 256
_F8 = 8


_INV_SQRT_HD = np.float32(1.0 / np.sqrt(_HD))


def _attn(h, hT, wqT, wk, wv, ws):
    f32 = jnp.float32
    qT = jnp.dot(wqT, hT, preferred_element_type=f32)
    k = jnp.dot(h, wk, preferred_element_type=f32)
    v = jnp.dot(h, wv, preferred_element_type=f32)
    s = jnp.dot(h, ws, preferred_element_type=f32)
    msgs = []
    for hh in range(_HEADS):
        k_h = k[:, hh * _HD:(hh + 1) * _HD]
        v_h = v[:, hh * _HD:(hh + 1) * _HD]
        groups = []
        for g in range(4):
            pairs = []
            for pr in range(4):
                d0 = g * 8 + pr * 2
                r0 = hh * _HD + d0
                c0 = k_h[:, d0:d0 + 1] * qT[r0:r0 + 1, :]
                c1 = k_h[:, d0 + 1:d0 + 2] * qT[r0 + 1:r0 + 2, :]
                pairs.append(c0 + c1)
            groups.append((pairs[0] + pairs[1]) + (pairs[2] + pairs[3]))
        lt = ((groups[0] + groups[1]) + groups[2]) + groups[3]
        lt = lt * _INV_SQRT_HD
        m = jnp.max(lt, axis=0, keepdims=True)
        exm = jnp.exp(lt - m)
        den = jnp.zeros((1, _T), f32)
        for j in range(_T):
            den = den + exm[j:j + 1, :]
        alpha = exm / (den + 1e-16)
        msg = jnp.zeros((_T, _HD), f32)
        for j in range(_T):
            aj = alpha[j:j + 1, :].reshape(_T, 1)
            msg = msg + aj * v_h[j:j + 1, :]
        msgs.append(msg)
    return jnp.concatenate(msgs, axis=1) + s


def _backbone_body(xT_ref, wmb_ref, bmb_ref, wq0T_ref, wk0_ref, wv0_ref,
                   ws0_ref, wq1T_ref, wk1_ref, wv1_ref, ws1_ref, wo_ref,
                   bo_ref, pwb_ref, rec_ref, score_ref):
    f32 = jnp.float32
    bf = jnp.bfloat16
    xT = xT_ref[0].astype(bf).astype(f32)
    wmb = wmb_ref[...].astype(bf).astype(f32)
    p0 = wmb[0] * xT[0:1, :]
    p1 = wmb[1] * xT[1:2, :]
    p2 = wmb[2] * xT[2:3, :]
    h0T = ((p0 + p1) + p2) + bmb_ref[...]
    h = h0T.T
    h = jax.nn.relu(_attn(h, h0T, wq0T_ref[...], wk0_ref[...], wv0_ref[...],
                          ws0_ref[...]))
    h = jax.nn.relu(_attn(h, h.T, wq1T_ref[...], wk1_ref[...], wv1_ref[...],
                          ws1_ref[...]))
    hout = jnp.dot(h, wo_ref[...], preferred_element_type=f32) + bo_ref[...]
    houtT = hout.T.astype(bf).astype(f32)
    prod = houtT * pwb_ref[...].astype(bf).astype(f32)
    groups = []
    for g in range(4):
        rows = [prod[g * 8 + r:g * 8 + r + 1, :] for r in range(8)]
        groups.append(((rows[0] + rows[1]) + (rows[2] + rows[3]))
                      + ((rows[4] + rows[5]) + (rows[6] + rows[7])))
    tot = ((groups[0] + groups[1]) + groups[2]) + groups[3]
    score_ref[0] = jnp.tanh(tot * rec_ref[...])


def _backbone(xT3, wmb, bmb, wq0T, wk0, wv0, ws0, wq1T, wk1, wv1, ws1,
              wo, bo2, pwb, rec2):
    full = lambda shape: pl.BlockSpec(shape, lambda i: (0,) * len(shape))
    return pl.pallas_call(
        _backbone_body,
        grid=(_PIECES,),
        in_specs=[
            pl.BlockSpec((1, _F8, _T), lambda i: (i, 0, 0)),
            full((3, 32, _T)), full((32, _T)),
            full((_HID, 32)), full((32, _HID)), full((32, _HID)),
            full((32, _HID)),
            full((_HID, _HID)), full((_HID, _HID)), full((_HID, _HID)),
            full((_HID, _HID)),
            full((_HID, 32)), full((1, 32)), full((32, _T)), full((1, 1)),
        ],
        out_specs=pl.BlockSpec((1, 1, _T), lambda i: (i, 0, 0)),
        out_shape=jax.ShapeDtypeStruct((_PIECES, 1, _T), jnp.float32),
    )(xT3, wmb, bmb, wq0T, wk0, wv0, ws0, wq1T, wk1, wv1, ws1, wo, bo2,
      pwb, rec2)


def _selector_body(score_hbm, xy_hbm, perm_o, val_o, xsel_o, esrc_o, edst_o,
                   sc_v, xy_v, tmp16_i, tmp16_f, inv_v, permg_v, valg_v,
                   xsel_v, esrc_v, edst_v, red_v):
    wid = lax.axis_index("s") * 2 + lax.axis_index("c")

    @pl.when(wid < _PIECES)
    def _():
        p = wid
        pltpu.sync_copy(score_hbm.at[pl.ds(p * _T, _T)], sc_v)
        pltpu.sync_copy(xy_hbm.at[pl.ds(p * _T * _F8, _T * _F8)], xy_v)
        lane = lax.iota(jnp.int32, 16)
        cur = [sc_v[pl.ds(j * 16, 16)] for j in range(4)]
        perm_acc = jnp.zeros((16,), jnp.int32)
        val_acc = jnp.zeros((16,), jnp.float32)
        for q in range(_KSEL):
            m = jnp.maximum(jnp.maximum(cur[0], cur[1]),
                            jnp.maximum(cur[2], cur[3]))
            for sh in (8, 4, 2, 1):
                red_v[...] = m
                m = jnp.maximum(m, plsc.load_gather(red_v, [lane ^ sh]))
            mval = m
            idxq = jnp.full((16,), 10**6, jnp.int32)
            for j in range(3, -1, -1):
                ffs = plsc.all_reduce_ffs(cur[j] == mval)
                idxq = jnp.where(ffs < 16, ffs + j * 16, idxq)
            perm_acc = jnp.where(lane == q, idxq, perm_acc)
            val_acc = jnp.where(lane == q, mval, val_acc)
            for j in range(4):
                cur[j] = jnp.where(lane + j * 16 == idxq, -2.0, cur[j])
        clamped = val_acc + (1.0 - val_acc)
        keys = jnp.where(lane < _KSEL, perm_acc, 1000)
        _, inv = plsc.sort_key_val(keys, lane)
        tmp16_i[...] = perm_acc
        tmp16_f[...] = clamped
        inv_v[...] = inv
        permg_v[...] = perm_acc + p * _T
        valg_v[...] = val_acc
        for vi in range(4):
            e = lane + vi * 16
            row = jnp.minimum(e // _F8, _KSEL - 1)
            col = e % _F8
            srow = plsc.load_gather(tmp16_i, [row])
            cl = plsc.load_gather(tmp16_f, [row])
            xv = plsc.load_gather(xy_v, [srow * _F8 + col])
            xsel_v[pl.ds(vi * 16, 16)] = xv * cl
        for vi in range(4):
            e = lane + vi * 16
            a = jnp.minimum(e // _KSEL, _KSEL - 1)
            b = e % _KSEL
            sa = plsc.load_gather(inv_v, [a])
            sb = plsc.load_gather(inv_v, [b])
            esrc_v[pl.ds(vi * 16, 16)] = sa + p * _KSEL
            edst_v[pl.ds(vi * 16, 16)] = sb + p * _KSEL
        pltpu.sync_copy(permg_v, perm_o.at[pl.ds(p * 16, 16)])
        pltpu.sync_copy(valg_v, val_o.at[pl.ds(p * 16, 16)])
        pltpu.sync_copy(xsel_v, xsel_o.at[pl.ds(p * 64, 64)])
        pltpu.sync_copy(esrc_v, esrc_o.at[pl.ds(p * 64, 64)])
        pltpu.sync_copy(edst_v, edst_o.at[pl.ds(p * 64, 64)])


@functools.cache
def _make_selector():
    return functools.partial(
        pl.kernel,
        out_type=[
        jax.ShapeDtypeStruct((_PIECES * 16,), jnp.int32),
        jax.ShapeDtypeStruct((_PIECES * 16,), jnp.float32),
        jax.ShapeDtypeStruct((_PIECES * 64,), jnp.float32),
            jax.ShapeDtypeStruct((_PIECES * 64,), jnp.int32),
            jax.ShapeDtypeStruct((_PIECES * 64,), jnp.int32),
        ],
        mesh=plsc.VectorSubcoreMesh(core_axis_name="c", subcore_axis_name="s",
                                    num_cores=2, num_subcores=16),
        compiler_params=pltpu.CompilerParams(needs_layout_passes=False),
        scratch_types=[
            pltpu.VMEM((_T,), jnp.float32),
            pltpu.VMEM((_T * _F8,), jnp.float32),
            pltpu.VMEM((16,), jnp.int32),
            pltpu.VMEM((16,), jnp.float32),
            pltpu.VMEM((16,), jnp.int32),
            pltpu.VMEM((16,), jnp.int32),
            pltpu.VMEM((16,), jnp.float32),
            pltpu.VMEM((_T,), jnp.float32),
            pltpu.VMEM((_T,), jnp.int32),
            pltpu.VMEM((_T,), jnp.int32),
            pltpu.VMEM((16,), jnp.float32),
        ],
    )(_selector_body)


def kernel(x, W_mlp, b_mlp, Wq0, Wk0, Wv0, Ws0, Wq1, Wk1, Wv1, Ws1,
           W_out, b_out, pool_w):
    B, K, F = x.shape
    xf = x.reshape(B * K, F)
    xy = jnp.pad(xf, ((0, 0), (0, _F8 - F)))
    xT3 = jnp.swapaxes(xy.reshape(_PIECES, _T, _F8), 1, 2)
    wmb = jnp.broadcast_to(W_mlp[:, :, None], (F, 32, _T))
    bmb = jnp.broadcast_to(b_mlp[:, None], (32, _T))
    pwb = jnp.broadcast_to(pool_w[:, None], (32, _T))
    rec2 = (1.0 / (jnp.linalg.norm(pool_w) + 1e-16)).reshape(1, 1)
    score3 = _backbone(xT3, wmb, bmb, Wq0.T, Wk0, Wv0, Ws0,
                       Wq1.T, Wk1, Wv1, Ws1, W_out, b_out.reshape(1, -1),
                       pwb, rec2)
    perm_o, val_o, xsel_o, esrc_o, edst_o = _make_selector()(
        score3.reshape(_PIECES * _T), xy.reshape(_PIECES * _T * _F8))
    perm = perm_o.reshape(_PIECES, 16)[:, :_KSEL].reshape(-1)
    score_sel = val_o.reshape(_PIECES, 16)[:, :_KSEL].reshape(-1)
    x_sel = xsel_o.reshape(_PIECES, _F8, _F8)[:, :_KSEL, :F].reshape(-1, F)
    edge_new = jnp.stack([
        esrc_o.reshape(_PIECES, 64)[:, :_KSEL * _KSEL].reshape(-1),
        edst_o.reshape(_PIECES, 64)[:, :_KSEL * _KSEL].reshape(-1)])
    new_batch = jnp.repeat(jnp.arange(_PIECES, dtype=jnp.int32), _KSEL)
    batch = jnp.repeat(jnp.arange(_PIECES, dtype=jnp.int32), _T)
    return (x_sel, perm, score_sel, edge_new, new_batch, batch)


# speedup vs baseline: 70.7092x; 1.3323x over previous
import functools

import numpy as np
import jax
import jax.numpy as jnp
from jax import lax
from jax.experimental import pallas as pl
from jax.experimental.pallas import tpu as pltpu
from jax.experimental.pallas import tpu_sc as plsc

_PIECES = 16
_T = 64
_KSEL = 7
_HEADS = 8
_HD = 32
_HID = 256
_F8 = 8


_INV_SQRT_HD = np.float32(1.0 / np.sqrt(_HD))


def _attn(h, hT, wqT, wk, wvT, wsT):
    f32 = jnp.float32
    qT = jnp.dot(wqT, hT, preferred_element_type=f32)
    k = jnp.dot(h, wk, preferred_element_type=f32)
    vT = jnp.dot(wvT, hT, preferred_element_type=f32)
    sT = jnp.dot(wsT, hT, preferred_element_type=f32)
    msgTs = []
    for hh in range(_HEADS):
        k_h = k[:, hh * _HD:(hh + 1) * _HD]
        vT_h = vT[hh * _HD:(hh + 1) * _HD, :]
        prods = [k_h[:, dd:dd + 1] * qT[hh * _HD + dd:hh * _HD + dd + 1, :]
                 for dd in range(_HD)]
        groups = []
        for g in range(4):
            pairs = [prods[g * 8 + 2 * pr] + prods[g * 8 + 2 * pr + 1]
                     for pr in range(4)]
            groups.append((pairs[0] + pairs[1]) + (pairs[2] + pairs[3]))
        lt = ((groups[0] + groups[1]) + groups[2]) + groups[3]
        lt = lt * _INV_SQRT_HD
        m = jnp.max(lt, axis=0, keepdims=True)
        exm = jnp.exp(lt - m)
        den = jnp.zeros((1, _T), f32)
        for j in range(_T):
            den = den + exm[j:j + 1, :]
        alpha = exm / (den + 1e-16)
        msgT = jnp.zeros((_HD, _T), f32)
        for j in range(_T):
            msgT = msgT + vT_h[:, j:j + 1] * alpha[j:j + 1, :]
        msgTs.append(msgT)
    return jnp.concatenate(msgTs, axis=0) + sT


def _backbone_body(xT_ref, wmb_ref, bmb_ref, wq0T_ref, wk0_ref, wv0T_ref,
                   ws0T_ref, wq1T_ref, wk1_ref, wv1T_ref, ws1T_ref, woT_ref,
                   bob_ref, pwb_ref, rec_ref, score_ref):
    f32 = jnp.float32
    bf = jnp.bfloat16
    xT = xT_ref[0].astype(bf).astype(f32)
    wmb = wmb_ref[...].astype(bf).astype(f32)
    p0 = wmb[0] * xT[0:1, :]
    p1 = wmb[1] * xT[1:2, :]
    p2 = wmb[2] * xT[2:3, :]
    h0T = ((p0 + p1) + p2) + bmb_ref[...]
    h0 = h0T.T
    h1T = jax.nn.relu(_attn(h0, h0T, wq0T_ref[...], wk0_ref[...],
                            wv0T_ref[...], ws0T_ref[...]))
    h2T = jax.nn.relu(_attn(h1T.T, h1T, wq1T_ref[...], wk1_ref[...],
                            wv1T_ref[...], ws1T_ref[...]))
    houtT = jnp.dot(woT_ref[...], h2T,
                    preferred_element_type=f32) + bob_ref[...]
    houtT = houtT.astype(bf).astype(f32)
    prod = houtT * pwb_ref[...].astype(bf).astype(f32)
    groups = []
    for g in range(4):
        rows = [prod[g * 8 + r:g * 8 + r + 1, :] for r in range(8)]
        groups.append(((rows[0] + rows[1]) + (rows[2] + rows[3]))
                      + ((rows[4] + rows[5]) + (rows[6] + rows[7])))
    tot = ((groups[0] + groups[1]) + groups[2]) + groups[3]
    score_ref[0] = jnp.tanh(tot * rec_ref[...])


def _backbone(xT3, wmb, bmb, wq0T, wk0, wv0T, ws0T, wq1T, wk1, wv1T, ws1T,
              woT, bob, pwb, rec2):
    full = lambda shape: pl.BlockSpec(shape, lambda i: (0,) * len(shape))
    return pl.pallas_call(
        _backbone_body,
        grid=(_PIECES,),
        in_specs=[
            pl.BlockSpec((1, _F8, _T), lambda i: (i, 0, 0)),
            full((3, 32, _T)), full((32, _T)),
            full((_HID, 32)), full((32, _HID)), full((_HID, 32)),
            full((_HID, 32)),
            full((_HID, _HID)), full((_HID, _HID)), full((_HID, _HID)),
            full((_HID, _HID)),
            full((32, _HID)), full((32, _T)), full((32, _T)), full((1, 1)),
        ],
        out_specs=pl.BlockSpec((1, 1, _T), lambda i: (i, 0, 0)),
        out_shape=jax.ShapeDtypeStruct((_PIECES, 1, _T), jnp.float32),
    )(xT3, wmb, bmb, wq0T, wk0, wv0T, ws0T, wq1T, wk1, wv1T, ws1T, woT, bob,
      pwb, rec2)


def _selector_body(score_hbm, xy_hbm, perm_o, val_o, xsel_o, esrc_o, edst_o,
                   sc_v, xy_v, tmp16_i, tmp16_f, inv_v, permg_v, valg_v,
                   xsel_v, esrc_v, edst_v, red_v):
    wid = lax.axis_index("s") * 2 + lax.axis_index("c")

    @pl.when(wid < _PIECES)
    def _():
        p = wid
        pltpu.sync_copy(score_hbm.at[pl.ds(p * _T, _T)], sc_v)
        pltpu.sync_copy(xy_hbm.at[pl.ds(p * _T * _F8, _T * _F8)], xy_v)
        lane = lax.iota(jnp.int32, 16)
        cur = [sc_v[pl.ds(j * 16, 16)] for j in range(4)]
        perm_acc = jnp.zeros((16,), jnp.int32)
        val_acc = jnp.zeros((16,), jnp.float32)
        for q in range(_KSEL):
            m = jnp.maximum(jnp.maximum(cur[0], cur[1]),
                            jnp.maximum(cur[2], cur[3]))
            for sh in (8, 4, 2, 1):
                red_v[...] = m
                m = jnp.maximum(m, plsc.load_gather(red_v, [lane ^ sh]))
            mval = m
            idxq = jnp.full((16,), 10**6, jnp.int32)
            for j in range(3, -1, -1):
                ffs = plsc.all_reduce_ffs(cur[j] == mval)
                idxq = jnp.where(ffs < 16, ffs + j * 16, idxq)
            perm_acc = jnp.where(lane == q, idxq, perm_acc)
            val_acc = jnp.where(lane == q, mval, val_acc)
            for j in range(4):
                cur[j] = jnp.where(lane + j * 16 == idxq, -2.0, cur[j])
        clamped = val_acc + (1.0 - val_acc)
        keys = jnp.where(lane < _KSEL, perm_acc, 1000)
        _, inv = plsc.sort_key_val(keys, lane)
        tmp16_i[...] = perm_acc
        tmp16_f[...] = clamped
        inv_v[...] = inv
        permg_v[...] = perm_acc + p * _T
        valg_v[...] = val_acc
        for vi in range(4):
            e = lane + vi * 16
            row = jnp.minimum(e // _F8, _KSEL - 1)
            col = e % _F8
            srow = plsc.load_gather(tmp16_i, [row])
            cl = plsc.load_gather(tmp16_f, [row])
            xv = plsc.load_gather(xy_v, [srow * _F8 + col])
            xsel_v[pl.ds(vi * 16, 16)] = xv * cl
        for vi in range(4):
            e = lane + vi * 16
            a = jnp.minimum(e // _KSEL, _KSEL - 1)
            b = e % _KSEL
            sa = plsc.load_gather(inv_v, [a])
            sb = plsc.load_gather(inv_v, [b])
            esrc_v[pl.ds(vi * 16, 16)] = sa + p * _KSEL
            edst_v[pl.ds(vi * 16, 16)] = sb + p * _KSEL
        pltpu.sync_copy(permg_v, perm_o.at[pl.ds(p * 16, 16)])
        pltpu.sync_copy(valg_v, val_o.at[pl.ds(p * 16, 16)])
        pltpu.sync_copy(xsel_v, xsel_o.at[pl.ds(p * 64, 64)])
        pltpu.sync_copy(esrc_v, esrc_o.at[pl.ds(p * 64, 64)])
        pltpu.sync_copy(edst_v, edst_o.at[pl.ds(p * 64, 64)])


@functools.cache
def _make_selector():
    return functools.partial(
        pl.kernel,
        out_type=[
        jax.ShapeDtypeStruct((_PIECES * 16,), jnp.int32),
        jax.ShapeDtypeStruct((_PIECES * 16,), jnp.float32),
        jax.ShapeDtypeStruct((_PIECES * 64,), jnp.float32),
            jax.ShapeDtypeStruct((_PIECES * 64,), jnp.int32),
            jax.ShapeDtypeStruct((_PIECES * 64,), jnp.int32),
        ],
        mesh=plsc.VectorSubcoreMesh(core_axis_name="c", subcore_axis_name="s",
                                    num_cores=2, num_subcores=16),
        compiler_params=pltpu.CompilerParams(needs_layout_passes=False),
        scratch_types=[
            pltpu.VMEM((_T,), jnp.float32),
            pltpu.VMEM((_T * _F8,), jnp.float32),
            pltpu.VMEM((16,), jnp.int32),
            pltpu.VMEM((16,), jnp.float32),
            pltpu.VMEM((16,), jnp.int32),
            pltpu.VMEM((16,), jnp.int32),
            pltpu.VMEM((16,), jnp.float32),
            pltpu.VMEM((_T,), jnp.float32),
            pltpu.VMEM((_T,), jnp.int32),
            pltpu.VMEM((_T,), jnp.int32),
            pltpu.VMEM((16,), jnp.float32),
        ],
    )(_selector_body)


def kernel(x, W_mlp, b_mlp, Wq0, Wk0, Wv0, Ws0, Wq1, Wk1, Wv1, Ws1,
           W_out, b_out, pool_w):
    B, K, F = x.shape
    xf = x.reshape(B * K, F)
    xy = jnp.pad(xf, ((0, 0), (0, _F8 - F)))
    xT3 = jnp.swapaxes(xy.reshape(_PIECES, _T, _F8), 1, 2)
    wmb = jnp.broadcast_to(W_mlp[:, :, None], (F, 32, _T))
    bmb = jnp.broadcast_to(b_mlp[:, None], (32, _T))
    pwb = jnp.broadcast_to(pool_w[:, None], (32, _T))
    rec2 = (1.0 / (jnp.linalg.norm(pool_w) + 1e-16)).reshape(1, 1)
    bob = jnp.broadcast_to(b_out[:, None], (32, _T))
    score3 = _backbone(xT3, wmb, bmb, Wq0.T, Wk0, Wv0.T, Ws0.T,
                       Wq1.T, Wk1, Wv1.T, Ws1.T, W_out.T, bob, pwb, rec2)
    perm_o, val_o, xsel_o, esrc_o, edst_o = _make_selector()(
        score3.reshape(_PIECES * _T), xy.reshape(_PIECES * _T * _F8))
    perm = perm_o.reshape(_PIECES, 16)[:, :_KSEL].reshape(-1)
    score_sel = val_o.reshape(_PIECES, 16)[:, :_KSEL].reshape(-1)
    x_sel = xsel_o.reshape(_PIECES, _F8, _F8)[:, :_KSEL, :F].reshape(-1, F)
    edge_new = jnp.stack([
        esrc_o.reshape(_PIECES, 64)[:, :_KSEL * _KSEL].reshape(-1),
        edst_o.reshape(_PIECES, 64)[:, :_KSEL * _KSEL].reshape(-1)])
    new_batch = jnp.repeat(jnp.arange(_PIECES, dtype=jnp.int32), _KSEL)
    batch = jnp.repeat(jnp.arange(_PIECES, dtype=jnp.int32), _T)
    return (x_sel, perm, score_sel, edge_new, new_batch, batch)
